# baseline (speedup 1.0000x reference)
import jax
import jax.numpy as jnp
from jax.experimental import pallas as pl

_B_BLK = 64


def _concat_body(x_ref, t_ref, o_ref):
    xb = x_ref[...]
    emb = t_ref[...]
    bB, L, D = xb.shape
    o_ref[:, :, :D] = xb
    o_ref[:, :, D:] = jnp.broadcast_to(emb[None], (bB, L, emb.shape[-1]))


def kernel(x, table):
    B, L, D = x.shape
    E = table.shape[-1]
    return pl.pallas_call(
        _concat_body,
        grid=(B // _B_BLK,),
        in_specs=[
            pl.BlockSpec((_B_BLK, L, D), lambda i: (i, 0, 0)),
            pl.BlockSpec((L, E), lambda i: (0, 0)),
        ],
        out_specs=pl.BlockSpec((_B_BLK, L, D + E), lambda i: (i, 0, 0)),
        out_shape=jax.ShapeDtypeStruct((B, L, D + E), x.dtype),
    )(x, table)
